# Initial kernel scaffold; baseline (speedup 1.0000x reference)
import functools

import jax
import jax.numpy as jnp
from jax import lax
from jax.experimental import pallas as pl
from jax.experimental.pallas import tpu as pltpu
from jax.experimental.pallas import tpu_sc as plsc

B = 16
P = 16800
G = 32
CH = 1680
NCH = P // CH
VPC = CH // 16
THR = 0.35
LN2 = 0.6931471805599453
SQRT2 = 1.4142135381698608

OFF_PR = 0
OFF_LOC = 4 * CH
OFF_CONF = 8 * CH
OFF_LM = 10 * CH
FBUF_W = 18 * CH

OFF_AREA = 416


def _splatf(x):
    return jnp.full((16,), x, jnp.float32)


def _splati(x):
    return jnp.full((16,), x, jnp.int32)


def _ln(x):
    bits = lax.bitcast_convert_type(x, jnp.int32)
    e = (bits >> 23) - 127
    m = lax.bitcast_convert_type((bits & 0x7FFFFF) | 0x3F800000, jnp.float32)
    big = m > SQRT2
    m = jnp.where(big, m * 0.5, m)
    e = e + jnp.where(big, 1, 0)
    z = (m - 1.0) / (m + 1.0)
    z2 = z * z
    p = jnp.float32(1.0 / 9.0)
    p = p * z2 + jnp.float32(1.0 / 7.0)
    p = p * z2 + jnp.float32(0.2)
    p = p * z2 + jnp.float32(1.0 / 3.0)
    p = p * z2 + jnp.float32(1.0)
    return e.astype(jnp.float32) * LN2 + 2.0 * z * p


def _smooth_l1(pred, tgt):
    d = pred - tgt
    ad = jnp.abs(d)
    return jnp.where(ad < 1.0, 0.5 * d * d, ad - 0.5)


def _make_sc_kernel():
    mesh = plsc.VectorSubcoreMesh(core_axis_name="c", subcore_axis_name="s")

    @functools.partial(
        pl.kernel,
        mesh=mesh,
        out_type=jax.ShapeDtypeStruct((B, 16), jnp.float32),
        scratch_types=[
            pltpu.VMEM((2, FBUF_W), jnp.float32),
            pltpu.VMEM((P,), jnp.float32),
            pltpu.VMEM((P,), jnp.int32),
            pltpu.VMEM((P,), jnp.int32),
            pltpu.VMEM((448,), jnp.float32),
            pltpu.VMEM((G * 16,), jnp.float32),
            pltpu.VMEM((G * 16,), jnp.int32),
            pltpu.VMEM((16,), jnp.float32),
            pltpu.SemaphoreType.DMA,
            pltpu.SemaphoreType.DMA,
            pltpu.SemaphoreType.DMA,
        ],
    )
    def sc_loss(priors_f, loc_f, conf_f, landm_f, targ_f, out,
                fbuf, bto_v, bti_v, mined_v, traw, rowmax, rowidx, outbuf,
                sem0, sem1, sem2):
        cix = lax.axis_index("c")
        six = lax.axis_index("s")
        wid = six * 2 + cix

        @pl.when(wid < B)
        def _body():
            b = wid
            ivec = lax.iota(jnp.int32, 16)
            i4 = ivec * 4
            i2 = ivec * 2
            i8 = ivec * 8
            i13 = ivec * 13
            sems = [sem0, sem1]

            pltpu.async_copy(targ_f.at[b], traw.at[pl.ds(0, 416)], sem2).wait()
            for g in range(2):
                gb = g * 208
                tx1 = plsc.load_gather(traw, [i13 + _splati(gb + 0)])
                ty1 = plsc.load_gather(traw, [i13 + _splati(gb + 1)])
                tx2 = plsc.load_gather(traw, [i13 + _splati(gb + 2)])
                ty2 = plsc.load_gather(traw, [i13 + _splati(gb + 3)])
                traw[pl.ds(OFF_AREA + g * 16, 16)] = (tx2 - tx1) * (ty2 - ty1)

            neg1 = _splatf(-1.0)
            zeroi = _splati(0)
            zerof = _splatf(0.0)
            for t in range(G):
                rowmax[pl.ds(t * 16, 16)] = neg1
                rowidx[pl.ds(t * 16, 16)] = zeroi

            def issue_p1(ci, slot):
                return [pltpu.async_copy(
                    priors_f.at[pl.ds(ci * CH * 4, CH * 4)],
                    fbuf.at[slot, pl.ds(OFF_PR, CH * 4)], sems[slot])]

            pend = [issue_p1(0, 0), None]
            for ci in range(NCH):
                slot = ci % 2
                for h in pend[slot]:
                    h.wait()
                if ci + 1 < NCH:
                    pend[1 - slot] = issue_p1(ci + 1, 1 - slot)
                base = ci * CH
                slotv = _splati(slot * FBUF_W)

                def p1_body(v, _, slotv=slotv, base=base):
                    o4 = v * 64
                    pcx = plsc.load_gather(fbuf, [slotv + i4 + jnp.full((16,), o4 + 0, jnp.int32)])
                    pcy = plsc.load_gather(fbuf, [slotv + i4 + jnp.full((16,), o4 + 1, jnp.int32)])
                    pw = plsc.load_gather(fbuf, [slotv + i4 + jnp.full((16,), o4 + 2, jnp.int32)])
                    ph = plsc.load_gather(fbuf, [slotv + i4 + jnp.full((16,), o4 + 3, jnp.int32)])
                    px1 = pcx - pw * 0.5
                    py1 = pcy - ph * 0.5
                    px2 = pcx + pw * 0.5
                    py2 = pcy + ph * 0.5
                    pa = (px2 - px1) * (py2 - py1)
                    pidx = ivec + (base + v * 16)

                    def t_body(t, carry):
                        btoc, btic = carry
                        t13 = t * 13
                        tx1 = plsc.load_gather(traw, [jnp.full((16,), t13 + 0, jnp.int32)])
                        ty1 = plsc.load_gather(traw, [jnp.full((16,), t13 + 1, jnp.int32)])
                        tx2 = plsc.load_gather(traw, [jnp.full((16,), t13 + 2, jnp.int32)])
                        ty2 = plsc.load_gather(traw, [jnp.full((16,), t13 + 3, jnp.int32)])
                        ta = plsc.load_gather(traw, [jnp.full((16,), OFF_AREA + t, jnp.int32)])
                        iw = jnp.maximum(jnp.minimum(tx2, px2) - jnp.maximum(tx1, px1), 0.0)
                        ih = jnp.maximum(jnp.minimum(ty2, py2) - jnp.maximum(ty1, py1), 0.0)
                        inter = iw * ih
                        ov = inter / (ta + pa - inter)
                        upd = ov > btoc
                        btoc = jnp.where(upd, ov, btoc)
                        btic = jnp.where(upd, jnp.full((16,), t, jnp.int32), btic)
                        rm = rowmax[pl.ds(t * 16, 16)]
                        ri = rowidx[pl.ds(t * 16, 16)]
                        u2 = ov > rm
                        rowmax[pl.ds(t * 16, 16)] = jnp.where(u2, ov, rm)
                        rowidx[pl.ds(t * 16, 16)] = jnp.where(u2, pidx, ri)
                        return btoc, btic

                    btoc, btic = lax.fori_loop(0, G, t_body, (neg1, zeroi))
                    bto_v[pl.ds(base + v * 16, 16)] = btoc
                    bti_v[pl.ds(base + v * 16, 16)] = btic
                    return 0

                lax.fori_loop(0, VPC, p1_body, 0)

            lane0 = ivec == 0
            two = _splatf(2.0)
            bigP = _splati(P)

            def force_body(t, _):
                rm = rowmax[pl.ds(t * 16, 16)]
                ri = rowidx[pl.ds(t * 16, 16)]
                mx = jnp.max(rm)
                idx = jnp.min(jnp.where(rm == mx, ri, bigP))
                iv = jnp.full((16,), idx, jnp.int32)
                plsc.store_scatter(bto_v, [iv], two, mask=lane0)
                plsc.store_scatter(bti_v, [iv], jnp.full((16,), t, jnp.int32), mask=lane0)
                return 0

            lax.fori_loop(0, G, force_body, 0)

            def issue_p2(ci, slot):
                d = CH * 4
                return [
                    pltpu.async_copy(priors_f.at[pl.ds(ci * d, d)],
                                     fbuf.at[slot, pl.ds(OFF_PR, d)], sems[slot]),
                    pltpu.async_copy(loc_f.at[b, pl.ds(ci * d, d)],
                                     fbuf.at[slot, pl.ds(OFF_LOC, d)], sems[slot]),
                    pltpu.async_copy(conf_f.at[b, pl.ds(ci * CH * 2, CH * 2)],
                                     fbuf.at[slot, pl.ds(OFF_CONF, CH * 2)], sems[slot]),
                    pltpu.async_copy(landm_f.at[b, pl.ds(ci * CH * 8, CH * 8)],
                                     fbuf.at[slot, pl.ds(OFF_LM, CH * 8)], sems[slot]),
                ]

            accs = (zerof, zerof, zerof, zeroi)
            pend = [issue_p2(0, 0), None]
            for ci in range(NCH):
                slot = ci % 2
                for h in pend[slot]:
                    h.wait()
                if ci + 1 < NCH:
                    pend[1 - slot] = issue_p2(ci + 1, 1 - slot)
                base = ci * CH
                slotv = _splati(slot * FBUF_W)

                def p2_body(v, carry, slotv=slotv, base=base):
                    al, am, ac, an = carry
                    o4 = v * 64
                    o2 = v * 32
                    o8 = v * 128
                    gsl = pl.ds(base + v * 16, 16)
                    pcx = plsc.load_gather(fbuf, [slotv + i4 + jnp.full((16,), o4 + 0, jnp.int32)])
                    pcy = plsc.load_gather(fbuf, [slotv + i4 + jnp.full((16,), o4 + 1, jnp.int32)])
                    pw = plsc.load_gather(fbuf, [slotv + i4 + jnp.full((16,), o4 + 2, jnp.int32)])
                    ph = plsc.load_gather(fbuf, [slotv + i4 + jnp.full((16,), o4 + 3, jnp.int32)])
                    bto = bto_v[gsl]
                    bti = bti_v[gsl]
                    pos = bto >= THR
                    posf = jnp.where(pos, 1.0, 0.0)
                    bt13 = bti * 13
                    tx1 = plsc.load_gather(traw, [bt13 + _splati(0)])
                    ty1 = plsc.load_gather(traw, [bt13 + _splati(1)])
                    tx2 = plsc.load_gather(traw, [bt13 + _splati(2)])
                    ty2 = plsc.load_gather(traw, [bt13 + _splati(3)])
                    dx = pw * 0.1
                    dy = ph * 0.1
                    g0 = ((tx1 + tx2) * 0.5 - pcx) / dx
                    g1 = ((ty1 + ty2) * 0.5 - pcy) / dy
                    g2 = _ln((tx2 - tx1) / pw) * 5.0
                    g3 = _ln((ty2 - ty1) / ph) * 5.0
                    l0 = plsc.load_gather(fbuf, [slotv + i4 + jnp.full((16,), OFF_LOC + o4 + 0, jnp.int32)])
                    l1 = plsc.load_gather(fbuf, [slotv + i4 + jnp.full((16,), OFF_LOC + o4 + 1, jnp.int32)])
                    l2 = plsc.load_gather(fbuf, [slotv + i4 + jnp.full((16,), OFF_LOC + o4 + 2, jnp.int32)])
                    l3 = plsc.load_gather(fbuf, [slotv + i4 + jnp.full((16,), OFF_LOC + o4 + 3, jnp.int32)])
                    al = al + (_smooth_l1(l0, g0) + _smooth_l1(l1, g1)
                               + _smooth_l1(l2, g2) + _smooth_l1(l3, g3)) * posf
                    lacc = zerof
                    for j in range(8):
                        lmp = plsc.load_gather(traw, [bt13 + _splati(4 + j)])
                        if j % 2 == 0:
                            gl = (lmp - pcx) / dx
                        else:
                            gl = (lmp - pcy) / dy
                        ld = plsc.load_gather(fbuf, [slotv + i8 + jnp.full((16,), OFF_LM + o8 + j, jnp.int32)])
                        lacc = lacc + _smooth_l1(ld, gl)
                    am = am + lacc * posf
                    c0 = plsc.load_gather(fbuf, [slotv + i2 + jnp.full((16,), OFF_CONF + o2 + 0, jnp.int32)])
                    c1 = plsc.load_gather(fbuf, [slotv + i2 + jnp.full((16,), OFF_CONF + o2 + 1, jnp.int32)])
                    mx = jnp.maximum(c0, c1)
                    mn = jnp.minimum(c0, c1)
                    lse = mx + _ln(1.0 + jnp.exp(mn - mx))
                    raw = lse - jnp.where(pos, c1, c0)
                    ac = ac + raw * posf
                    an = an + jnp.where(pos, 1, 0)
                    mined_v[gsl] = lax.bitcast_convert_type(
                        jnp.where(pos, 0.0, raw), jnp.int32)
                    return al, am, ac, an

                accs = lax.fori_loop(0, VPC, p2_body, accs)
            al, am, ac, an = accs

            npos = jnp.sum(an)
            k = jnp.minimum(npos * 7, P - 1)

            def count_gt(x):
                xs = jnp.full((16,), x, jnp.int32)

                def cb(i, acc):
                    for u in range(10):
                        vv = mined_v[pl.ds((i * 10 + u) * 16, 16)]
                        acc = acc + jnp.where(vv > xs, 1, 0)
                    return acc

                return jnp.sum(lax.fori_loop(0, P // 160, cb, zeroi))

            f0 = count_gt(0)

            def bis(i, ans):
                cand = ans | (1 << (30 - i))
                return jnp.where(count_gt(cand) >= k, cand, ans)

            ans = lax.fori_loop(0, 31, bis, 0)
            tbits = jnp.where(f0 >= k, ans + 1, 0)
            ts_ = jnp.full((16,), tbits, jnp.int32)

            def fin(i, carry):
                sacc, cacc = carry
                for u in range(10):
                    vv = mined_v[pl.ds((i * 10 + u) * 16, 16)]
                    mm = vv > ts_
                    val = lax.bitcast_convert_type(vv, jnp.float32)
                    sacc = sacc + jnp.where(mm, val, 0.0)
                    cacc = cacc + jnp.where(mm, 1, 0)
                return sacc, cacc

            sacc, cacc = lax.fori_loop(0, P // 160, fin, (zerof, zeroi))
            tval = jnp.max(lax.bitcast_convert_type(ts_, jnp.float32))
            topk = jnp.sum(sacc) + (k - jnp.sum(cacc)).astype(jnp.float32) * tval
            loss_c = jnp.sum(ac) + topk

            outv = jnp.where(ivec == 0, jnp.sum(al),
                   jnp.where(ivec == 1, loss_c,
                   jnp.where(ivec == 2, jnp.sum(am), npos.astype(jnp.float32))))
            outbuf[...] = outv
            pltpu.sync_copy(outbuf, out.at[b])

    return sc_loss


_SC_LOSS = _make_sc_kernel()


def kernel(loc_data, conf_data, landm_data, priors, targets):
    Bn, Pn, _ = loc_data.shape
    priors_f = priors.reshape(Pn * 4)
    loc_f = loc_data.reshape(Bn, Pn * 4)
    conf_f = conf_data.reshape(Bn, Pn * 2)
    landm_f = landm_data.reshape(Bn, Pn * 8)
    targ_f = targets.reshape(Bn, targets.shape[1] * targets.shape[2])
    part = _SC_LOSS(priors_f, loc_f, conf_f, landm_f, targ_f)
    n = jnp.maximum(jnp.sum(part[:, 3]), 1.0)
    return part[:, 0].sum() / n, part[:, 1].sum() / n, part[:, 2].sum() / n


# speedup vs baseline: 13.1444x; 13.1444x over previous
import functools

import jax
import jax.numpy as jnp
from jax import lax
from jax.experimental import pallas as pl
from jax.experimental.pallas import tpu as pltpu
from jax.experimental.pallas import tpu_sc as plsc

B = 16
P = 16800
G = 32
CH = 1680
NCH = P // CH
VPC = CH // 16
THR = 0.35
LN2 = 0.6931471805599453
SQRT2 = 1.4142135381698608

FBUF_W = 18 * CH
OFF_AREA = 416


def _splatf(x):
    return jnp.full((16,), x, jnp.float32)


def _splati(x):
    return jnp.full((16,), x, jnp.int32)


def _ln(x):
    bits = lax.bitcast_convert_type(x, jnp.int32)
    e = (bits >> 23) - 127
    m = lax.bitcast_convert_type((bits & 0x7FFFFF) | 0x3F800000, jnp.float32)
    big = m > SQRT2
    m = jnp.where(big, m * 0.5, m)
    e = e + jnp.where(big, 1, 0)
    z = (m - 1.0) / (m + 1.0)
    z2 = z * z
    p = jnp.float32(1.0 / 9.0)
    p = p * z2 + jnp.float32(1.0 / 7.0)
    p = p * z2 + jnp.float32(0.2)
    p = p * z2 + jnp.float32(1.0 / 3.0)
    p = p * z2 + jnp.float32(1.0)
    return e.astype(jnp.float32) * LN2 + 2.0 * z * p


def _smooth_l1(pred, tgt):
    d = pred - tgt
    ad = jnp.abs(d)
    return jnp.where(ad < 1.0, 0.5 * d * d, ad - 0.5)


def _make_sc_kernel():
    mesh = plsc.VectorSubcoreMesh(core_axis_name="c", subcore_axis_name="s")

    @functools.partial(
        pl.kernel,
        mesh=mesh,
        out_type=jax.ShapeDtypeStruct((B, 16), jnp.float32),
        compiler_params=pltpu.CompilerParams(
            use_tc_tiling_on_sc=False, needs_layout_passes=False),
        scratch_types=[
            pltpu.VMEM((FBUF_W,), jnp.float32),
            pltpu.VMEM((FBUF_W,), jnp.float32),
            pltpu.VMEM((P,), jnp.float32),
            pltpu.VMEM((P,), jnp.int32),
            pltpu.VMEM((P,), jnp.int32),
            pltpu.VMEM((448,), jnp.float32),
            pltpu.VMEM((G * 16,), jnp.float32),
            pltpu.VMEM((G * 16,), jnp.int32),
            pltpu.VMEM((16,), jnp.float32),
            pltpu.SemaphoreType.DMA,
            pltpu.SemaphoreType.DMA,
            pltpu.SemaphoreType.DMA,
        ],
    )
    def sc_loss(priors_f, loc_f, conf_f, landm_f, targ_f, out,
                fbuf0, fbuf1, bto_v, bti_v, mined_v, traw, rowmax, rowidx,
                outbuf, sem0, sem1, sem2):
        cix = lax.axis_index("c")
        six = lax.axis_index("s")
        wid = six * 2 + cix

        @pl.when(wid < B)
        def _body():
            b = wid
            ivec = lax.iota(jnp.int32, 16)
            i13 = ivec * 13
            sems = [sem0, sem1]
            fbufs = [fbuf0, fbuf1]

            pltpu.async_copy(targ_f.at[pl.ds(b * 416, 416)],
                             traw.at[pl.ds(0, 416)], sem2).wait()
            for g in range(2):
                gb = g * 208
                tx1 = plsc.load_gather(traw, [i13 + _splati(gb + 0)])
                ty1 = plsc.load_gather(traw, [i13 + _splati(gb + 1)])
                tx2 = plsc.load_gather(traw, [i13 + _splati(gb + 2)])
                ty2 = plsc.load_gather(traw, [i13 + _splati(gb + 3)])
                traw[pl.ds(OFF_AREA + g * 16, 16)] = (tx2 - tx1) * (ty2 - ty1)

            neg1 = _splatf(-1.0)
            zeroi = _splati(0)
            zerof = _splatf(0.0)
            for t in range(G):
                rowmax[pl.ds(t * 16, 16)] = neg1
                rowidx[pl.ds(t * 16, 16)] = zeroi

            def issue_p1(ci, slot):
                return [pltpu.async_copy(
                    priors_f.at[pl.ds(p * P + ci * CH, CH)],
                    fbufs[slot].at[pl.ds(p * CH, CH)], sems[slot])
                    for p in range(4)]

            pend = [issue_p1(0, 0), None]
            for ci in range(NCH):
                slot = ci % 2
                for h in pend[slot]:
                    h.wait()
                if ci + 1 < NCH:
                    pend[1 - slot] = issue_p1(ci + 1, 1 - slot)
                base = ci * CH
                fb = fbufs[slot]

                def p1_body(v, _, fb=fb, base=base):
                    off = v * 16
                    pcx = fb[pl.ds(0 * CH + off, 16)]
                    pcy = fb[pl.ds(1 * CH + off, 16)]
                    pw = fb[pl.ds(2 * CH + off, 16)]
                    ph = fb[pl.ds(3 * CH + off, 16)]
                    px1 = pcx - pw * 0.5
                    py1 = pcy - ph * 0.5
                    px2 = pcx + pw * 0.5
                    py2 = pcy + ph * 0.5
                    pa = (px2 - px1) * (py2 - py1)
                    pidx = ivec + (base + off)

                    def t_body(t, carry):
                        btoc, btic = carry
                        t13 = t * 13
                        tx1 = plsc.load_gather(traw, [jnp.full((16,), t13 + 0, jnp.int32)])
                        ty1 = plsc.load_gather(traw, [jnp.full((16,), t13 + 1, jnp.int32)])
                        tx2 = plsc.load_gather(traw, [jnp.full((16,), t13 + 2, jnp.int32)])
                        ty2 = plsc.load_gather(traw, [jnp.full((16,), t13 + 3, jnp.int32)])
                        ta = plsc.load_gather(traw, [jnp.full((16,), OFF_AREA + t, jnp.int32)])
                        iw = jnp.maximum(jnp.minimum(tx2, px2) - jnp.maximum(tx1, px1), 0.0)
                        ih = jnp.maximum(jnp.minimum(ty2, py2) - jnp.maximum(ty1, py1), 0.0)
                        inter = iw * ih
                        ov = inter / (ta + pa - inter)
                        upd = ov > btoc
                        btoc = jnp.where(upd, ov, btoc)
                        btic = jnp.where(upd, jnp.full((16,), t, jnp.int32), btic)
                        rm = rowmax[pl.ds(t * 16, 16)]
                        ri = rowidx[pl.ds(t * 16, 16)]
                        u2 = ov > rm
                        rowmax[pl.ds(t * 16, 16)] = jnp.where(u2, ov, rm)
                        rowidx[pl.ds(t * 16, 16)] = jnp.where(u2, pidx, ri)
                        return btoc, btic

                    btoc, btic = lax.fori_loop(0, G, t_body, (neg1, zeroi))
                    bto_v[pl.ds(base + off, 16)] = btoc
                    bti_v[pl.ds(base + off, 16)] = btic
                    return 0

                lax.fori_loop(0, VPC, p1_body, 0)

            lane0 = ivec == 0
            two = _splatf(2.0)
            bigP = _splati(P)

            def force_body(t, _):
                rm = rowmax[pl.ds(t * 16, 16)]
                ri = rowidx[pl.ds(t * 16, 16)]
                mx = jnp.max(rm)
                idx = jnp.min(jnp.where(rm == mx, ri, bigP))
                iv = jnp.full((16,), idx, jnp.int32)
                plsc.store_scatter(bto_v, [iv], two, mask=lane0)
                plsc.store_scatter(bti_v, [iv], jnp.full((16,), t, jnp.int32), mask=lane0)
                return 0

            lax.fori_loop(0, G, force_body, 0)

            def issue_p2(ci, slot):
                fs = fbufs[slot]
                hs = [pltpu.async_copy(
                    priors_f.at[pl.ds(p * P + ci * CH, CH)],
                    fs.at[pl.ds(p * CH, CH)], sems[slot]) for p in range(4)]
                hs += [pltpu.async_copy(
                    loc_f.at[pl.ds(b * (4 * P) + j * P + ci * CH, CH)],
                    fs.at[pl.ds((4 + j) * CH, CH)], sems[slot]) for j in range(4)]
                hs += [pltpu.async_copy(
                    conf_f.at[pl.ds(b * (2 * P) + j * P + ci * CH, CH)],
                    fs.at[pl.ds((8 + j) * CH, CH)], sems[slot]) for j in range(2)]
                hs += [pltpu.async_copy(
                    landm_f.at[pl.ds(b * (8 * P) + j * P + ci * CH, CH)],
                    fs.at[pl.ds((10 + j) * CH, CH)], sems[slot]) for j in range(8)]
                return hs

            accs = (zerof, zerof, zerof, zeroi)
            pend = [issue_p2(0, 0), None]
            for ci in range(NCH):
                slot = ci % 2
                for h in pend[slot]:
                    h.wait()
                if ci + 1 < NCH:
                    pend[1 - slot] = issue_p2(ci + 1, 1 - slot)
                base = ci * CH
                fb = fbufs[slot]

                def p2_body(v, carry, fb=fb, base=base):
                    al, am, ac, an = carry
                    off = v * 16
                    gsl = pl.ds(base + off, 16)
                    pcx = fb[pl.ds(0 * CH + off, 16)]
                    pcy = fb[pl.ds(1 * CH + off, 16)]
                    pw = fb[pl.ds(2 * CH + off, 16)]
                    ph = fb[pl.ds(3 * CH + off, 16)]
                    bto = bto_v[gsl]
                    bti = bti_v[gsl]
                    pos = bto >= THR
                    posf = jnp.where(pos, 1.0, 0.0)
                    bt13 = bti * 13
                    tx1 = plsc.load_gather(traw, [bt13 + _splati(0)])
                    ty1 = plsc.load_gather(traw, [bt13 + _splati(1)])
                    tx2 = plsc.load_gather(traw, [bt13 + _splati(2)])
                    ty2 = plsc.load_gather(traw, [bt13 + _splati(3)])
                    dx = pw * 0.1
                    dy = ph * 0.1
                    g0 = ((tx1 + tx2) * 0.5 - pcx) / dx
                    g1 = ((ty1 + ty2) * 0.5 - pcy) / dy
                    g2 = _ln((tx2 - tx1) / pw) * 5.0
                    g3 = _ln((ty2 - ty1) / ph) * 5.0
                    al = al + (_smooth_l1(fb[pl.ds(4 * CH + off, 16)], g0)
                               + _smooth_l1(fb[pl.ds(5 * CH + off, 16)], g1)
                               + _smooth_l1(fb[pl.ds(6 * CH + off, 16)], g2)
                               + _smooth_l1(fb[pl.ds(7 * CH + off, 16)], g3)) * posf
                    lacc = zerof
                    for j in range(8):
                        lmp = plsc.load_gather(traw, [bt13 + _splati(4 + j)])
                        if j % 2 == 0:
                            gl = (lmp - pcx) / dx
                        else:
                            gl = (lmp - pcy) / dy
                        lacc = lacc + _smooth_l1(fb[pl.ds((10 + j) * CH + off, 16)], gl)
                    am = am + lacc * posf
                    c0 = fb[pl.ds(8 * CH + off, 16)]
                    c1 = fb[pl.ds(9 * CH + off, 16)]
                    mx = jnp.maximum(c0, c1)
                    mn = jnp.minimum(c0, c1)
                    lse = mx + _ln(1.0 + jnp.exp(mn - mx))
                    raw = lse - jnp.where(pos, c1, c0)
                    ac = ac + raw * posf
                    an = an + jnp.where(pos, 1, 0)
                    mined_v[gsl] = lax.bitcast_convert_type(
                        jnp.where(pos, 0.0, raw), jnp.int32)
                    return al, am, ac, an

                accs = lax.fori_loop(0, VPC, p2_body, accs)
            al, am, ac, an = accs

            npos = jnp.sum(an)
            k = jnp.minimum(npos * 7, P - 1)

            def count_gt(x):
                xs = jnp.full((16,), x, jnp.int32)

                def cb(i, acc):
                    for u in range(10):
                        vv = mined_v[pl.ds((i * 10 + u) * 16, 16)]
                        acc = acc + jnp.where(vv > xs, 1, 0)
                    return acc

                return jnp.sum(lax.fori_loop(0, P // 160, cb, zeroi))

            f0 = count_gt(0)

            def bis(i, ans):
                cand = ans | (1 << (30 - i))
                return jnp.where(count_gt(cand) >= k, cand, ans)

            ans = lax.fori_loop(0, 31, bis, 0)
            tbits = jnp.where(f0 >= k, ans + 1, 0)
            ts_ = jnp.full((16,), tbits, jnp.int32)

            def fin(i, carry):
                sacc, cacc = carry
                for u in range(10):
                    vv = mined_v[pl.ds((i * 10 + u) * 16, 16)]
                    mm = vv > ts_
                    val = lax.bitcast_convert_type(vv, jnp.float32)
                    sacc = sacc + jnp.where(mm, val, 0.0)
                    cacc = cacc + jnp.where(mm, 1, 0)
                return sacc, cacc

            sacc, cacc = lax.fori_loop(0, P // 160, fin, (zerof, zeroi))
            tval = jnp.max(lax.bitcast_convert_type(ts_, jnp.float32))
            topk = jnp.sum(sacc) + (k - jnp.sum(cacc)).astype(jnp.float32) * tval
            loss_c = jnp.sum(ac) + topk

            outv = jnp.where(ivec == 0, jnp.sum(al),
                   jnp.where(ivec == 1, loss_c,
                   jnp.where(ivec == 2, jnp.sum(am), npos.astype(jnp.float32))))
            outbuf[...] = outv
            pltpu.sync_copy(outbuf, out.at[b])

    return sc_loss


_SC_LOSS = _make_sc_kernel()


def kernel(loc_data, conf_data, landm_data, priors, targets):
    Bn, Pn, _ = loc_data.shape
    priors_f = priors.T.reshape(-1)
    loc_f = loc_data.transpose(0, 2, 1).reshape(-1)
    conf_f = conf_data.transpose(0, 2, 1).reshape(-1)
    landm_f = landm_data.transpose(0, 2, 1).reshape(-1)
    targ_f = targets.reshape(-1)
    part = _SC_LOSS(priors_f, loc_f, conf_f, landm_f, targ_f)
    n = jnp.maximum(jnp.sum(part[:, 3]), 1.0)
    return part[:, 0].sum() / n, part[:, 1].sum() / n, part[:, 2].sum() / n


# speedup vs baseline: 13.2307x; 1.0066x over previous
import functools

import jax
import jax.numpy as jnp
from jax import lax
from jax.experimental import pallas as pl
from jax.experimental.pallas import tpu as pltpu
from jax.experimental.pallas import tpu_sc as plsc

B = 16
P = 16800
G = 32
CH = 1680
NCH = P // CH
VPC = CH // 16
THR = 0.35
LN2 = 0.6931471805599453
SQRT2 = 1.4142135381698608

FBUF_W = 18 * CH
OFF_AREA = 416


def _splatf(x):
    return jnp.full((16,), x, jnp.float32)


def _splati(x):
    return jnp.full((16,), x, jnp.int32)


def _ln(x):
    bits = lax.bitcast_convert_type(x, jnp.int32)
    e = (bits >> 23) - 127
    m = lax.bitcast_convert_type((bits & 0x7FFFFF) | 0x3F800000, jnp.float32)
    big = m > SQRT2
    m = jnp.where(big, m * 0.5, m)
    e = e + jnp.where(big, 1, 0)
    z = (m - 1.0) / (m + 1.0)
    z2 = z * z
    p = jnp.float32(1.0 / 9.0)
    p = p * z2 + jnp.float32(1.0 / 7.0)
    p = p * z2 + jnp.float32(0.2)
    p = p * z2 + jnp.float32(1.0 / 3.0)
    p = p * z2 + jnp.float32(1.0)
    return e.astype(jnp.float32) * LN2 + 2.0 * z * p


def _smooth_l1(pred, tgt):
    d = pred - tgt
    ad = jnp.abs(d)
    return jnp.where(ad < 1.0, 0.5 * d * d, ad - 0.5)


def _make_sc_kernel():
    mesh = plsc.VectorSubcoreMesh(core_axis_name="c", subcore_axis_name="s")

    @functools.partial(
        pl.kernel,
        mesh=mesh,
        out_type=jax.ShapeDtypeStruct((B, 16), jnp.float32),
        compiler_params=pltpu.CompilerParams(
            use_tc_tiling_on_sc=False, needs_layout_passes=False),
        scratch_types=[
            pltpu.VMEM((FBUF_W,), jnp.float32),
            pltpu.VMEM((FBUF_W,), jnp.float32),
            pltpu.VMEM((P,), jnp.float32),
            pltpu.VMEM((P,), jnp.int32),
            pltpu.VMEM((P,), jnp.int32),
            pltpu.VMEM((448,), jnp.float32),
            pltpu.VMEM((G * 16,), jnp.float32),
            pltpu.VMEM((G * 16,), jnp.int32),
            pltpu.VMEM((16,), jnp.float32),
            pltpu.SemaphoreType.DMA,
            pltpu.SemaphoreType.DMA,
            pltpu.SemaphoreType.DMA,
        ],
    )
    def sc_loss(priors_f, loc_f, conf_f, landm_f, targ_f, out,
                fbuf0, fbuf1, bto_v, bti_v, mined_v, traw, rowmax, rowidx,
                outbuf, sem0, sem1, sem2):
        cix = lax.axis_index("c")
        six = lax.axis_index("s")
        wid = six * 2 + cix

        @pl.when(wid < B)
        def _body():
            b = wid
            ivec = lax.iota(jnp.int32, 16)
            i13 = ivec * 13
            sems = [sem0, sem1]
            fbufs = [fbuf0, fbuf1]

            pltpu.async_copy(targ_f.at[pl.ds(b * 416, 416)],
                             traw.at[pl.ds(0, 416)], sem2).wait()
            for g in range(2):
                gb = g * 208
                tx1 = plsc.load_gather(traw, [i13 + _splati(gb + 0)])
                ty1 = plsc.load_gather(traw, [i13 + _splati(gb + 1)])
                tx2 = plsc.load_gather(traw, [i13 + _splati(gb + 2)])
                ty2 = plsc.load_gather(traw, [i13 + _splati(gb + 3)])
                traw[pl.ds(OFF_AREA + g * 16, 16)] = (tx2 - tx1) * (ty2 - ty1)

            neg1 = _splatf(-1.0)
            zeroi = _splati(0)
            zerof = _splatf(0.0)
            for t in range(G):
                rowmax[pl.ds(t * 16, 16)] = neg1
                rowidx[pl.ds(t * 16, 16)] = zeroi

            def issue_p1(ci, slot):
                return [pltpu.async_copy(
                    priors_f.at[pl.ds(p * P + ci * CH, CH)],
                    fbufs[slot].at[pl.ds(p * CH, CH)], sems[slot])
                    for p in range(4)]

            pend = [issue_p1(0, 0), None]
            for ci in range(NCH):
                slot = ci % 2
                for h in pend[slot]:
                    h.wait()
                if ci + 1 < NCH:
                    pend[1 - slot] = issue_p1(ci + 1, 1 - slot)
                base = ci * CH
                fb = fbufs[slot]

                def pf_body(v, _, fb=fb, base=base):
                    off = v * 16
                    pcx = fb[pl.ds(0 * CH + off, 16)]
                    pcy = fb[pl.ds(1 * CH + off, 16)]
                    pw = fb[pl.ds(2 * CH + off, 16)]
                    ph = fb[pl.ds(3 * CH + off, 16)]
                    px1 = pcx - pw * 0.5
                    py1 = pcy - ph * 0.5
                    px2 = pcx + pw * 0.5
                    py2 = pcy + ph * 0.5
                    fb[pl.ds(4 * CH + off, 16)] = px1
                    fb[pl.ds(5 * CH + off, 16)] = py1
                    fb[pl.ds(6 * CH + off, 16)] = px2
                    fb[pl.ds(7 * CH + off, 16)] = py2
                    fb[pl.ds(8 * CH + off, 16)] = (px2 - px1) * (py2 - py1)
                    bto_v[pl.ds(base + off, 16)] = neg1
                    bti_v[pl.ds(base + off, 16)] = zeroi
                    return 0

                lax.fori_loop(0, VPC, pf_body, 0)

                def t_body(t, _, fb=fb, base=base):
                    t13 = t * 13
                    tx1 = plsc.load_gather(traw, [jnp.full((16,), t13 + 0, jnp.int32)])
                    ty1 = plsc.load_gather(traw, [jnp.full((16,), t13 + 1, jnp.int32)])
                    tx2 = plsc.load_gather(traw, [jnp.full((16,), t13 + 2, jnp.int32)])
                    ty2 = plsc.load_gather(traw, [jnp.full((16,), t13 + 3, jnp.int32)])
                    ta = plsc.load_gather(traw, [jnp.full((16,), OFF_AREA + t, jnp.int32)])
                    tsplat = jnp.full((16,), t, jnp.int32)
                    rm = rowmax[pl.ds(t * 16, 16)]
                    ri = rowidx[pl.ds(t * 16, 16)]

                    def v_body(v, carry, fb=fb, base=base):
                        rm, ri = carry
                        for u in range(3):
                            off = (v * 3 + u) * 16
                            px1 = fb[pl.ds(4 * CH + off, 16)]
                            py1 = fb[pl.ds(5 * CH + off, 16)]
                            px2 = fb[pl.ds(6 * CH + off, 16)]
                            py2 = fb[pl.ds(7 * CH + off, 16)]
                            pa = fb[pl.ds(8 * CH + off, 16)]
                            iw = jnp.maximum(jnp.minimum(tx2, px2) - jnp.maximum(tx1, px1), 0.0)
                            ih = jnp.maximum(jnp.minimum(ty2, py2) - jnp.maximum(ty1, py1), 0.0)
                            inter = iw * ih
                            ov = inter / (ta + pa - inter)
                            bto = bto_v[pl.ds(base + off, 16)]
                            bti = bti_v[pl.ds(base + off, 16)]
                            upd = ov > bto
                            bto_v[pl.ds(base + off, 16)] = jnp.where(upd, ov, bto)
                            bti_v[pl.ds(base + off, 16)] = jnp.where(upd, tsplat, bti)
                            u2 = ov > rm
                            rm = jnp.where(u2, ov, rm)
                            ri = jnp.where(u2, ivec + (base + off), ri)
                        return rm, ri

                    rm, ri = lax.fori_loop(0, VPC // 3, v_body, (rm, ri))
                    rowmax[pl.ds(t * 16, 16)] = rm
                    rowidx[pl.ds(t * 16, 16)] = ri
                    return 0

                lax.fori_loop(0, G, t_body, 0)

            lane0 = ivec == 0
            two = _splatf(2.0)
            bigP = _splati(P)

            def force_body(t, _):
                rm = rowmax[pl.ds(t * 16, 16)]
                ri = rowidx[pl.ds(t * 16, 16)]
                mx = jnp.max(rm)
                idx = jnp.min(jnp.where(rm == mx, ri, bigP))
                iv = jnp.full((16,), idx, jnp.int32)
                plsc.store_scatter(bto_v, [iv], two, mask=lane0)
                plsc.store_scatter(bti_v, [iv], jnp.full((16,), t, jnp.int32), mask=lane0)
                return 0

            lax.fori_loop(0, G, force_body, 0)

            def issue_p2(ci, slot):
                fs = fbufs[slot]
                hs = [pltpu.async_copy(
                    priors_f.at[pl.ds(p * P + ci * CH, CH)],
                    fs.at[pl.ds(p * CH, CH)], sems[slot]) for p in range(4)]
                hs += [pltpu.async_copy(
                    loc_f.at[pl.ds(b * (4 * P) + j * P + ci * CH, CH)],
                    fs.at[pl.ds((4 + j) * CH, CH)], sems[slot]) for j in range(4)]
                hs += [pltpu.async_copy(
                    conf_f.at[pl.ds(b * (2 * P) + j * P + ci * CH, CH)],
                    fs.at[pl.ds((8 + j) * CH, CH)], sems[slot]) for j in range(2)]
                hs += [pltpu.async_copy(
                    landm_f.at[pl.ds(b * (8 * P) + j * P + ci * CH, CH)],
                    fs.at[pl.ds((10 + j) * CH, CH)], sems[slot]) for j in range(8)]
                return hs

            accs = (zerof, zerof, zerof, zeroi)
            pend = [issue_p2(0, 0), None]
            for ci in range(NCH):
                slot = ci % 2
                for h in pend[slot]:
                    h.wait()
                if ci + 1 < NCH:
                    pend[1 - slot] = issue_p2(ci + 1, 1 - slot)
                base = ci * CH
                fb = fbufs[slot]

                def p2_body(v, carry, fb=fb, base=base):
                    al, am, ac, an = carry
                    off = v * 16
                    gsl = pl.ds(base + off, 16)
                    pcx = fb[pl.ds(0 * CH + off, 16)]
                    pcy = fb[pl.ds(1 * CH + off, 16)]
                    pw = fb[pl.ds(2 * CH + off, 16)]
                    ph = fb[pl.ds(3 * CH + off, 16)]
                    bto = bto_v[gsl]
                    bti = bti_v[gsl]
                    pos = bto >= THR
                    posf = jnp.where(pos, 1.0, 0.0)
                    bt13 = bti * 13
                    tx1 = plsc.load_gather(traw, [bt13 + _splati(0)])
                    ty1 = plsc.load_gather(traw, [bt13 + _splati(1)])
                    tx2 = plsc.load_gather(traw, [bt13 + _splati(2)])
                    ty2 = plsc.load_gather(traw, [bt13 + _splati(3)])
                    dx = pw * 0.1
                    dy = ph * 0.1
                    g0 = ((tx1 + tx2) * 0.5 - pcx) / dx
                    g1 = ((ty1 + ty2) * 0.5 - pcy) / dy
                    g2 = _ln((tx2 - tx1) / pw) * 5.0
                    g3 = _ln((ty2 - ty1) / ph) * 5.0
                    al = al + (_smooth_l1(fb[pl.ds(4 * CH + off, 16)], g0)
                               + _smooth_l1(fb[pl.ds(5 * CH + off, 16)], g1)
                               + _smooth_l1(fb[pl.ds(6 * CH + off, 16)], g2)
                               + _smooth_l1(fb[pl.ds(7 * CH + off, 16)], g3)) * posf
                    lacc = zerof
                    for j in range(8):
                        lmp = plsc.load_gather(traw, [bt13 + _splati(4 + j)])
                        if j % 2 == 0:
                            gl = (lmp - pcx) / dx
                        else:
                            gl = (lmp - pcy) / dy
                        lacc = lacc + _smooth_l1(fb[pl.ds((10 + j) * CH + off, 16)], gl)
                    am = am + lacc * posf
                    c0 = fb[pl.ds(8 * CH + off, 16)]
                    c1 = fb[pl.ds(9 * CH + off, 16)]
                    mx = jnp.maximum(c0, c1)
                    mn = jnp.minimum(c0, c1)
                    lse = mx + _ln(1.0 + jnp.exp(mn - mx))
                    raw = lse - jnp.where(pos, c1, c0)
                    ac = ac + raw * posf
                    an = an + jnp.where(pos, 1, 0)
                    mined_v[gsl] = lax.bitcast_convert_type(
                        jnp.where(pos, 0.0, raw), jnp.int32)
                    return al, am, ac, an

                accs = lax.fori_loop(0, VPC, p2_body, accs)
            al, am, ac, an = accs

            npos = jnp.sum(an)
            k = jnp.minimum(npos * 7, P - 1)

            def count_gt(x):
                xs = jnp.full((16,), x, jnp.int32)

                def cb(i, acc):
                    for u in range(10):
                        vv = mined_v[pl.ds((i * 10 + u) * 16, 16)]
                        acc = acc + jnp.where(vv > xs, 1, 0)
                    return acc

                return jnp.sum(lax.fori_loop(0, P // 160, cb, zeroi))

            f0 = count_gt(0)

            def bis(i, ans):
                cand = ans | (1 << (30 - i))
                return jnp.where(count_gt(cand) >= k, cand, ans)

            ans = lax.fori_loop(0, 31, bis, 0)
            tbits = jnp.where(f0 >= k, ans + 1, 0)
            ts_ = jnp.full((16,), tbits, jnp.int32)

            def fin(i, carry):
                sacc, cacc = carry
                for u in range(10):
                    vv = mined_v[pl.ds((i * 10 + u) * 16, 16)]
                    mm = vv > ts_
                    val = lax.bitcast_convert_type(vv, jnp.float32)
                    sacc = sacc + jnp.where(mm, val, 0.0)
                    cacc = cacc + jnp.where(mm, 1, 0)
                return sacc, cacc

            sacc, cacc = lax.fori_loop(0, P // 160, fin, (zerof, zeroi))
            tval = jnp.max(lax.bitcast_convert_type(ts_, jnp.float32))
            topk = jnp.sum(sacc) + (k - jnp.sum(cacc)).astype(jnp.float32) * tval
            loss_c = jnp.sum(ac) + topk

            outv = jnp.where(ivec == 0, jnp.sum(al),
                   jnp.where(ivec == 1, loss_c,
                   jnp.where(ivec == 2, jnp.sum(am), npos.astype(jnp.float32))))
            outbuf[...] = outv
            pltpu.sync_copy(outbuf, out.at[b])

    return sc_loss


_SC_LOSS = _make_sc_kernel()


def kernel(loc_data, conf_data, landm_data, priors, targets):
    Bn, Pn, _ = loc_data.shape
    priors_f = priors.T.reshape(-1)
    loc_f = loc_data.transpose(0, 2, 1).reshape(-1)
    conf_f = conf_data.transpose(0, 2, 1).reshape(-1)
    landm_f = landm_data.transpose(0, 2, 1).reshape(-1)
    targ_f = targets.reshape(-1)
    part = _SC_LOSS(priors_f, loc_f, conf_f, landm_f, targ_f)
    n = jnp.maximum(jnp.sum(part[:, 3]), 1.0)
    return part[:, 0].sum() / n, part[:, 1].sum() / n, part[:, 2].sum() / n


# speedup vs baseline: 24.8546x; 1.8786x over previous
import functools

import jax
import jax.numpy as jnp
from jax import lax
from jax.experimental import pallas as pl
from jax.experimental.pallas import tpu as pltpu
from jax.experimental.pallas import tpu_sc as plsc

B = 16
P = 16800
G = 32
CH = 1680
NCH = P // CH
VPC = CH // 16
THR = 0.35
LN2 = 0.6931471805599453
SQRT2 = 1.4142135381698608

FBUF_W = 18 * CH
OFF_AREA = 416


def _splatf(x):
    return jnp.full((16,), x, jnp.float32)


def _splati(x):
    return jnp.full((16,), x, jnp.int32)


def _ln(x):
    bits = lax.bitcast_convert_type(x, jnp.int32)
    e = (bits >> 23) - 127
    m = lax.bitcast_convert_type((bits & 0x7FFFFF) | 0x3F800000, jnp.float32)
    big = m > SQRT2
    m = jnp.where(big, m * 0.5, m)
    e = e + jnp.where(big, 1, 0)
    z = (m - 1.0) / (m + 1.0)
    z2 = z * z
    p = jnp.float32(1.0 / 9.0)
    p = p * z2 + jnp.float32(1.0 / 7.0)
    p = p * z2 + jnp.float32(0.2)
    p = p * z2 + jnp.float32(1.0 / 3.0)
    p = p * z2 + jnp.float32(1.0)
    return e.astype(jnp.float32) * LN2 + 2.0 * z * p


def _smooth_l1(pred, tgt):
    d = pred - tgt
    ad = jnp.abs(d)
    return jnp.where(ad < 1.0, 0.5 * d * d, ad - 0.5)


def _make_sc_kernel():
    mesh = plsc.VectorSubcoreMesh(core_axis_name="c", subcore_axis_name="s")

    @functools.partial(
        pl.kernel,
        mesh=mesh,
        out_type=jax.ShapeDtypeStruct((B, 16), jnp.float32),
        compiler_params=pltpu.CompilerParams(
            use_tc_tiling_on_sc=False, needs_layout_passes=False),
        scratch_types=[
            pltpu.VMEM((FBUF_W,), jnp.float32),
            pltpu.VMEM((FBUF_W,), jnp.float32),
            pltpu.VMEM((P,), jnp.float32),
            pltpu.VMEM((P,), jnp.int32),
            pltpu.VMEM((P,), jnp.int32),
            pltpu.VMEM((448,), jnp.float32),
            pltpu.VMEM((G * 16,), jnp.float32),
            pltpu.VMEM((G * 16,), jnp.int32),
            pltpu.VMEM((16,), jnp.float32),
            pltpu.SemaphoreType.DMA,
            pltpu.SemaphoreType.DMA,
            pltpu.SemaphoreType.DMA,
        ],
    )
    def sc_loss(priors_f, loc_f, conf_f, landm_f, targ_f, out,
                fbuf0, fbuf1, bto_v, bti_v, mined_v, traw, rowmax, rowidx,
                outbuf, sem0, sem1, sem2):
        cix = lax.axis_index("c")
        six = lax.axis_index("s")
        wid = six * 2 + cix

        @pl.when(wid < B)
        def _body():
            b = wid
            ivec = lax.iota(jnp.int32, 16)
            i13 = ivec * 13
            sems = [sem0, sem1]
            fbufs = [fbuf0, fbuf1]

            pltpu.async_copy(targ_f.at[pl.ds(b * 416, 416)],
                             traw.at[pl.ds(0, 416)], sem2).wait()
            for g in range(2):
                gb = g * 208
                tx1 = plsc.load_gather(traw, [i13 + _splati(gb + 0)])
                ty1 = plsc.load_gather(traw, [i13 + _splati(gb + 1)])
                tx2 = plsc.load_gather(traw, [i13 + _splati(gb + 2)])
                ty2 = plsc.load_gather(traw, [i13 + _splati(gb + 3)])
                traw[pl.ds(OFF_AREA + g * 16, 16)] = (tx2 - tx1) * (ty2 - ty1)

            neg1 = _splatf(-1.0)
            zeroi = _splati(0)
            zerof = _splatf(0.0)
            for t in range(G):
                rowmax[pl.ds(t * 16, 16)] = neg1
                rowidx[pl.ds(t * 16, 16)] = zeroi

            def issue_p1(ci, slot):
                return [pltpu.async_copy(
                    priors_f.at[pl.ds(p * P + ci * CH, CH)],
                    fbufs[slot].at[pl.ds(p * CH, CH)], sems[slot])
                    for p in range(4)]

            pend = [issue_p1(0, 0), None]
            for ci in range(NCH):
                slot = ci % 2
                for h in pend[slot]:
                    h.wait()
                if ci + 1 < NCH:
                    pend[1 - slot] = issue_p1(ci + 1, 1 - slot)
                base = ci * CH
                fb = fbufs[slot]

                def pf_body(v, _, fb=fb, base=base):
                    off = v * 16
                    pcx = fb[pl.ds(0 * CH + off, 16)]
                    pcy = fb[pl.ds(1 * CH + off, 16)]
                    pw = fb[pl.ds(2 * CH + off, 16)]
                    ph = fb[pl.ds(3 * CH + off, 16)]
                    px1 = pcx - pw * 0.5
                    py1 = pcy - ph * 0.5
                    px2 = pcx + pw * 0.5
                    py2 = pcy + ph * 0.5
                    fb[pl.ds(4 * CH + off, 16)] = px1
                    fb[pl.ds(5 * CH + off, 16)] = py1
                    fb[pl.ds(6 * CH + off, 16)] = px2
                    fb[pl.ds(7 * CH + off, 16)] = py2
                    fb[pl.ds(8 * CH + off, 16)] = (px2 - px1) * (py2 - py1)
                    bto_v[pl.ds(base + off, 16)] = neg1
                    bti_v[pl.ds(base + off, 16)] = zeroi
                    return 0

                lax.fori_loop(0, VPC, pf_body, 0)

                def tb_body(tb, _, fb=fb, base=base):
                    t0 = tb * 8
                    tc = []
                    for j in range(8):
                        t13 = (t0 + j) * 13
                        tc.append((
                            plsc.load_gather(traw, [jnp.full((16,), t13 + 0, jnp.int32)]),
                            plsc.load_gather(traw, [jnp.full((16,), t13 + 1, jnp.int32)]),
                            plsc.load_gather(traw, [jnp.full((16,), t13 + 2, jnp.int32)]),
                            plsc.load_gather(traw, [jnp.full((16,), t13 + 3, jnp.int32)]),
                            plsc.load_gather(traw, [jnp.full((16,), OFF_AREA + t0 + j, jnp.int32)]),
                            jnp.full((16,), t0 + j, jnp.int32),
                        ))
                    rmri = []
                    for j in range(8):
                        rmri.append(rowmax[pl.ds((t0 + j) * 16, 16)])
                        rmri.append(rowidx[pl.ds((t0 + j) * 16, 16)])

                    def v_body(v, carry, fb=fb, base=base):
                        rmri = list(carry)
                        off = v * 16
                        px1 = fb[pl.ds(4 * CH + off, 16)]
                        py1 = fb[pl.ds(5 * CH + off, 16)]
                        px2 = fb[pl.ds(6 * CH + off, 16)]
                        py2 = fb[pl.ds(7 * CH + off, 16)]
                        pa = fb[pl.ds(8 * CH + off, 16)]
                        pidx = ivec + (base + off)
                        ovs = []
                        for j in range(8):
                            tx1, ty1, tx2, ty2, ta, _ts = tc[j]
                            iw = jnp.maximum(jnp.minimum(tx2, px2) - jnp.maximum(tx1, px1), 0.0)
                            ih = jnp.maximum(jnp.minimum(ty2, py2) - jnp.maximum(ty1, py1), 0.0)
                            inter = iw * ih
                            ovs.append(inter / (ta + pa - inter))
                        bto = bto_v[pl.ds(base + off, 16)]
                        bti = bti_v[pl.ds(base + off, 16)]
                        for j in range(8):
                            upd = ovs[j] > bto
                            bto = jnp.where(upd, ovs[j], bto)
                            bti = jnp.where(upd, tc[j][5], bti)
                            u2 = ovs[j] > rmri[2 * j]
                            rmri[2 * j] = jnp.where(u2, ovs[j], rmri[2 * j])
                            rmri[2 * j + 1] = jnp.where(u2, pidx, rmri[2 * j + 1])
                        bto_v[pl.ds(base + off, 16)] = bto
                        bti_v[pl.ds(base + off, 16)] = bti
                        return tuple(rmri)

                    rmri = lax.fori_loop(0, VPC, v_body, tuple(rmri))
                    for j in range(8):
                        rowmax[pl.ds((t0 + j) * 16, 16)] = rmri[2 * j]
                        rowidx[pl.ds((t0 + j) * 16, 16)] = rmri[2 * j + 1]
                    return 0

                lax.fori_loop(0, G // 8, tb_body, 0)

            lane0 = ivec == 0
            two = _splatf(2.0)
            bigP = _splati(P)

            def force_body(t, _):
                rm = rowmax[pl.ds(t * 16, 16)]
                ri = rowidx[pl.ds(t * 16, 16)]
                mx = jnp.max(rm)
                idx = jnp.min(jnp.where(rm == mx, ri, bigP))
                iv = jnp.full((16,), idx, jnp.int32)
                plsc.store_scatter(bto_v, [iv], two, mask=lane0)
                plsc.store_scatter(bti_v, [iv], jnp.full((16,), t, jnp.int32), mask=lane0)
                return 0

            lax.fori_loop(0, G, force_body, 0)

            def issue_p2(ci, slot):
                fs = fbufs[slot]
                hs = [pltpu.async_copy(
                    priors_f.at[pl.ds(p * P + ci * CH, CH)],
                    fs.at[pl.ds(p * CH, CH)], sems[slot]) for p in range(4)]
                hs += [pltpu.async_copy(
                    loc_f.at[pl.ds(b * (4 * P) + j * P + ci * CH, CH)],
                    fs.at[pl.ds((4 + j) * CH, CH)], sems[slot]) for j in range(4)]
                hs += [pltpu.async_copy(
                    conf_f.at[pl.ds(b * (2 * P) + j * P + ci * CH, CH)],
                    fs.at[pl.ds((8 + j) * CH, CH)], sems[slot]) for j in range(2)]
                hs += [pltpu.async_copy(
                    landm_f.at[pl.ds(b * (8 * P) + j * P + ci * CH, CH)],
                    fs.at[pl.ds((10 + j) * CH, CH)], sems[slot]) for j in range(8)]
                return hs

            accs = (zerof, zerof, zerof, zeroi)
            pend = [issue_p2(0, 0), None]
            for ci in range(NCH):
                slot = ci % 2
                for h in pend[slot]:
                    h.wait()
                if ci + 1 < NCH:
                    pend[1 - slot] = issue_p2(ci + 1, 1 - slot)
                base = ci * CH
                fb = fbufs[slot]

                def p2_body(v, carry, fb=fb, base=base):
                    al, am, ac, an = carry
                    off = v * 16
                    gsl = pl.ds(base + off, 16)
                    pcx = fb[pl.ds(0 * CH + off, 16)]
                    pcy = fb[pl.ds(1 * CH + off, 16)]
                    pw = fb[pl.ds(2 * CH + off, 16)]
                    ph = fb[pl.ds(3 * CH + off, 16)]
                    bto = bto_v[gsl]
                    bti = bti_v[gsl]
                    pos = bto >= THR
                    posf = jnp.where(pos, 1.0, 0.0)
                    bt13 = bti * 13
                    tx1 = plsc.load_gather(traw, [bt13 + _splati(0)])
                    ty1 = plsc.load_gather(traw, [bt13 + _splati(1)])
                    tx2 = plsc.load_gather(traw, [bt13 + _splati(2)])
                    ty2 = plsc.load_gather(traw, [bt13 + _splati(3)])
                    dx = pw * 0.1
                    dy = ph * 0.1
                    g0 = ((tx1 + tx2) * 0.5 - pcx) / dx
                    g1 = ((ty1 + ty2) * 0.5 - pcy) / dy
                    g2 = _ln((tx2 - tx1) / pw) * 5.0
                    g3 = _ln((ty2 - ty1) / ph) * 5.0
                    al = al + (_smooth_l1(fb[pl.ds(4 * CH + off, 16)], g0)
                               + _smooth_l1(fb[pl.ds(5 * CH + off, 16)], g1)
                               + _smooth_l1(fb[pl.ds(6 * CH + off, 16)], g2)
                               + _smooth_l1(fb[pl.ds(7 * CH + off, 16)], g3)) * posf
                    lacc = zerof
                    for j in range(8):
                        lmp = plsc.load_gather(traw, [bt13 + _splati(4 + j)])
                        if j % 2 == 0:
                            gl = (lmp - pcx) / dx
                        else:
                            gl = (lmp - pcy) / dy
                        lacc = lacc + _smooth_l1(fb[pl.ds((10 + j) * CH + off, 16)], gl)
                    am = am + lacc * posf
                    c0 = fb[pl.ds(8 * CH + off, 16)]
                    c1 = fb[pl.ds(9 * CH + off, 16)]
                    mx = jnp.maximum(c0, c1)
                    mn = jnp.minimum(c0, c1)
                    lse = mx + _ln(1.0 + jnp.exp(mn - mx))
                    raw = lse - jnp.where(pos, c1, c0)
                    ac = ac + raw * posf
                    an = an + jnp.where(pos, 1, 0)
                    mined_v[gsl] = lax.bitcast_convert_type(
                        jnp.where(pos, 0.0, raw), jnp.int32)
                    return al, am, ac, an

                accs = lax.fori_loop(0, VPC, p2_body, accs)
            al, am, ac, an = accs

            npos = jnp.sum(an)
            k = jnp.minimum(npos * 7, P - 1)

            def count_gt(x):
                xs = jnp.full((16,), x, jnp.int32)

                def cb(i, acc):
                    for u in range(10):
                        vv = mined_v[pl.ds((i * 10 + u) * 16, 16)]
                        acc = acc + jnp.where(vv > xs, 1, 0)
                    return acc

                return jnp.sum(lax.fori_loop(0, P // 160, cb, zeroi))

            f0 = count_gt(0)

            def bis(i, ans):
                cand = ans | (1 << (30 - i))
                return jnp.where(count_gt(cand) >= k, cand, ans)

            ans = lax.fori_loop(0, 31, bis, 0)
            tbits = jnp.where(f0 >= k, ans + 1, 0)
            ts_ = jnp.full((16,), tbits, jnp.int32)

            def fin(i, carry):
                sacc, cacc = carry
                for u in range(10):
                    vv = mined_v[pl.ds((i * 10 + u) * 16, 16)]
                    mm = vv > ts_
                    val = lax.bitcast_convert_type(vv, jnp.float32)
                    sacc = sacc + jnp.where(mm, val, 0.0)
                    cacc = cacc + jnp.where(mm, 1, 0)
                return sacc, cacc

            sacc, cacc = lax.fori_loop(0, P // 160, fin, (zerof, zeroi))
            tval = jnp.max(lax.bitcast_convert_type(ts_, jnp.float32))
            topk = jnp.sum(sacc) + (k - jnp.sum(cacc)).astype(jnp.float32) * tval
            loss_c = jnp.sum(ac) + topk

            outv = jnp.where(ivec == 0, jnp.sum(al),
                   jnp.where(ivec == 1, loss_c,
                   jnp.where(ivec == 2, jnp.sum(am), npos.astype(jnp.float32))))
            outbuf[...] = outv
            pltpu.sync_copy(outbuf, out.at[b])

    return sc_loss


_SC_LOSS = _make_sc_kernel()


def kernel(loc_data, conf_data, landm_data, priors, targets):
    Bn, Pn, _ = loc_data.shape
    priors_f = priors.T.reshape(-1)
    loc_f = loc_data.transpose(0, 2, 1).reshape(-1)
    conf_f = conf_data.transpose(0, 2, 1).reshape(-1)
    landm_f = landm_data.transpose(0, 2, 1).reshape(-1)
    targ_f = targets.reshape(-1)
    part = _SC_LOSS(priors_f, loc_f, conf_f, landm_f, targ_f)
    n = jnp.maximum(jnp.sum(part[:, 3]), 1.0)
    return part[:, 0].sum() / n, part[:, 1].sum() / n, part[:, 2].sum() / n


# speedup vs baseline: 49.0504x; 1.9735x over previous
import functools

import jax
import jax.numpy as jnp
from jax import lax
from jax.experimental import pallas as pl
from jax.experimental.pallas import tpu as pltpu
from jax.experimental.pallas import tpu_sc as plsc

B = 16
P = 16800
G = 32
HALF = P // 2
CH = 1680
NCHH = HALF // CH
VPC = CH // 16
THR = 0.35
LN2 = 0.6931471805599453
SQRT2 = 1.4142135381698608

FBUF_W = 18 * CH
OFF_AREA = 416
NSLOT = 40


def _splatf(x):
    return jnp.full((16,), x, jnp.float32)


def _splati(x):
    return jnp.full((16,), x, jnp.int32)


def _ln(x):
    bits = lax.bitcast_convert_type(x, jnp.int32)
    e = (bits >> 23) - 127
    m = lax.bitcast_convert_type((bits & 0x7FFFFF) | 0x3F800000, jnp.float32)
    big = m > SQRT2
    m = jnp.where(big, m * 0.5, m)
    e = e + jnp.where(big, 1, 0)
    z = (m - 1.0) / (m + 1.0)
    z2 = z * z
    p = jnp.float32(1.0 / 9.0)
    p = p * z2 + jnp.float32(1.0 / 7.0)
    p = p * z2 + jnp.float32(0.2)
    p = p * z2 + jnp.float32(1.0 / 3.0)
    p = p * z2 + jnp.float32(1.0)
    return e.astype(jnp.float32) * LN2 + 2.0 * z * p


def _smooth_l1(pred, tgt):
    d = pred - tgt
    ad = jnp.abs(d)
    return jnp.where(ad < 1.0, 0.5 * d * d, ad - 0.5)


def _make_sc_kernel():
    mesh = plsc.VectorSubcoreMesh(core_axis_name="c", subcore_axis_name="s")

    @functools.partial(
        pl.kernel,
        mesh=mesh,
        out_type=jax.ShapeDtypeStruct((2 * B, 16), jnp.float32),
        compiler_params=pltpu.CompilerParams(
            use_tc_tiling_on_sc=False, needs_layout_passes=False),
        scratch_types=[
            pltpu.VMEM((FBUF_W,), jnp.float32),
            pltpu.VMEM((FBUF_W,), jnp.float32),
            pltpu.VMEM((P,), jnp.float32),
            pltpu.VMEM((P,), jnp.int32),
            pltpu.VMEM((P,), jnp.int32),
            pltpu.VMEM((448,), jnp.float32),
            pltpu.VMEM((G * 16,), jnp.float32),
            pltpu.VMEM((G * 16,), jnp.int32),
            pltpu.VMEM((16,), jnp.float32),
            pltpu.VMEM((32,), jnp.float32),
            pltpu.VMEM((32,), jnp.int32),
            pltpu.VMEM((32,), jnp.float32),
            pltpu.VMEM((32,), jnp.float32),
            pltpu.VMEM((32,), jnp.int32),
            pltpu.VMEM((32,), jnp.int32),
            pltpu.VMEM((32,), jnp.int32),
            pltpu.VMEM((16,), jnp.int32),
            pltpu.VMEM((16,), jnp.int32),
            pltpu.VMEM_SHARED((16 * 32,), jnp.float32),
            pltpu.VMEM_SHARED((16 * 32,), jnp.int32),
            pltpu.VMEM_SHARED((16 * NSLOT * 16,), jnp.int32),
            pltpu.SemaphoreType.DMA,
            pltpu.SemaphoreType.DMA,
            pltpu.SemaphoreType.DMA,
        ],
    )
    def sc_loss(priors_f, loc_f, conf_f, landm_f, targ_f, out,
                fbuf0, fbuf1, bto_v, bti_v, mined_v, traw, rowmax, rowidx,
                outbuf, myf, myi, h0f, h1f, h0i, h1i, mrg, cout, cin,
                shf, shi, shc, sem0, sem1, sem2):
        cix = lax.axis_index("c")
        six = lax.axis_index("s")
        w = cix * 16 + six
        b = w // 2
        h = w % 2
        lo = h * HALF

        ivec = lax.iota(jnp.int32, 16)
        i13 = ivec * 13
        sems = [sem0, sem1]
        fbufs = [fbuf0, fbuf1]
        lane0 = ivec == 0
        neg1 = _splatf(-1.0)
        zeroi = _splati(0)
        zerof = _splatf(0.0)
        bigP = _splati(P)

        def xchg_count(val, it):
            cout[...] = jnp.full((16,), val, jnp.int32)
            pltpu.sync_copy(cout, shc.at[pl.ds((six * NSLOT + it) * 16, 16)])
            plsc.subcore_barrier()
            pltpu.sync_copy(shc.at[pl.ds(((six ^ 1) * NSLOT + it) * 16, 16)], cin)
            return jnp.max(cin[...])

        pltpu.async_copy(targ_f.at[pl.ds(b * 416, 416)],
                         traw.at[pl.ds(0, 416)], sem2).wait()
        for g in range(2):
            gb = g * 208
            tx1 = plsc.load_gather(traw, [i13 + _splati(gb + 0)])
            ty1 = plsc.load_gather(traw, [i13 + _splati(gb + 1)])
            tx2 = plsc.load_gather(traw, [i13 + _splati(gb + 2)])
            ty2 = plsc.load_gather(traw, [i13 + _splati(gb + 3)])
            traw[pl.ds(OFF_AREA + g * 16, 16)] = (tx2 - tx1) * (ty2 - ty1)

        for t in range(G):
            rowmax[pl.ds(t * 16, 16)] = neg1
            rowidx[pl.ds(t * 16, 16)] = zeroi

        def issue_p1(cl, slot):
            return [pltpu.async_copy(
                priors_f.at[pl.ds(p * P + lo + cl * CH, CH)],
                fbufs[slot].at[pl.ds(p * CH, CH)], sems[slot])
                for p in range(4)]

        pend = [issue_p1(0, 0), None]
        for cl in range(NCHH):
            slot = cl % 2
            for hh in pend[slot]:
                hh.wait()
            if cl + 1 < NCHH:
                pend[1 - slot] = issue_p1(cl + 1, 1 - slot)
            base = lo + cl * CH
            fb = fbufs[slot]

            def pf_body(v, _, fb=fb, base=base):
                off = v * 16
                pcx = fb[pl.ds(0 * CH + off, 16)]
                pcy = fb[pl.ds(1 * CH + off, 16)]
                pw = fb[pl.ds(2 * CH + off, 16)]
                ph = fb[pl.ds(3 * CH + off, 16)]
                px1 = pcx - pw * 0.5
                py1 = pcy - ph * 0.5
                px2 = pcx + pw * 0.5
                py2 = pcy + ph * 0.5
                fb[pl.ds(4 * CH + off, 16)] = px1
                fb[pl.ds(5 * CH + off, 16)] = py1
                fb[pl.ds(6 * CH + off, 16)] = px2
                fb[pl.ds(7 * CH + off, 16)] = py2
                fb[pl.ds(8 * CH + off, 16)] = (px2 - px1) * (py2 - py1)
                bto_v[pl.ds(base + off, 16)] = neg1
                bti_v[pl.ds(base + off, 16)] = zeroi
                return 0

            lax.fori_loop(0, VPC, pf_body, 0)

            def tb_body(tb, _, fb=fb, base=base):
                t0 = tb * 4
                tc = []
                for j in range(4):
                    t13 = (t0 + j) * 13
                    tc.append((
                        plsc.load_gather(traw, [jnp.full((16,), t13 + 0, jnp.int32)]),
                        plsc.load_gather(traw, [jnp.full((16,), t13 + 1, jnp.int32)]),
                        plsc.load_gather(traw, [jnp.full((16,), t13 + 2, jnp.int32)]),
                        plsc.load_gather(traw, [jnp.full((16,), t13 + 3, jnp.int32)]),
                        plsc.load_gather(traw, [jnp.full((16,), OFF_AREA + t0 + j, jnp.int32)]),
                        jnp.full((16,), t0 + j, jnp.int32),
                    ))
                rmri = []
                for j in range(4):
                    rmri.append(rowmax[pl.ds((t0 + j) * 16, 16)])
                    rmri.append(rowidx[pl.ds((t0 + j) * 16, 16)])

                def v_body(v, carry, fb=fb, base=base):
                    rmri = list(carry)
                    off = v * 16
                    px1 = fb[pl.ds(4 * CH + off, 16)]
                    py1 = fb[pl.ds(5 * CH + off, 16)]
                    px2 = fb[pl.ds(6 * CH + off, 16)]
                    py2 = fb[pl.ds(7 * CH + off, 16)]
                    pa = fb[pl.ds(8 * CH + off, 16)]
                    pidx = ivec + (base + off)
                    ovs = []
                    for j in range(4):
                        tx1, ty1, tx2, ty2, ta, _ts = tc[j]
                        iw = jnp.maximum(jnp.minimum(tx2, px2) - jnp.maximum(tx1, px1), 0.0)
                        ih = jnp.maximum(jnp.minimum(ty2, py2) - jnp.maximum(ty1, py1), 0.0)
                        inter = iw * ih
                        ovs.append(inter / (ta + pa - inter))
                    bto = bto_v[pl.ds(base + off, 16)]
                    bti = bti_v[pl.ds(base + off, 16)]
                    for j in range(4):
                        upd = ovs[j] > bto
                        bto = jnp.where(upd, ovs[j], bto)
                        bti = jnp.where(upd, tc[j][5], bti)
                        u2 = ovs[j] > rmri[2 * j]
                        rmri[2 * j] = jnp.where(u2, ovs[j], rmri[2 * j])
                        rmri[2 * j + 1] = jnp.where(u2, pidx, rmri[2 * j + 1])
                    bto_v[pl.ds(base + off, 16)] = bto
                    bti_v[pl.ds(base + off, 16)] = bti
                    return tuple(rmri)

                rmri = lax.fori_loop(0, VPC, v_body, tuple(rmri))
                for j in range(4):
                    rowmax[pl.ds((t0 + j) * 16, 16)] = rmri[2 * j]
                    rowidx[pl.ds((t0 + j) * 16, 16)] = rmri[2 * j + 1]
                return 0

            lax.fori_loop(0, G // 4, tb_body, 0)

        def red_body(t, _):
            rm = rowmax[pl.ds(t * 16, 16)]
            ri = rowidx[pl.ds(t * 16, 16)]
            mx = jnp.max(rm)
            idx = jnp.min(jnp.where(rm == mx, ri, bigP))
            tv = jnp.full((16,), t, jnp.int32)
            plsc.store_scatter(myf, [tv], jnp.full((16,), mx, jnp.float32), mask=lane0)
            plsc.store_scatter(myi, [tv], jnp.full((16,), idx, jnp.int32), mask=lane0)
            return 0

        lax.fori_loop(0, G, red_body, 0)
        pltpu.sync_copy(myf, shf.at[pl.ds(six * 32, 32)])
        pltpu.sync_copy(myi, shi.at[pl.ds(six * 32, 32)])
        plsc.subcore_barrier()
        even = six - h
        pltpu.sync_copy(shf.at[pl.ds(even * 32, 32)], h0f)
        pltpu.sync_copy(shf.at[pl.ds((even + 1) * 32, 32)], h1f)
        pltpu.sync_copy(shi.at[pl.ds(even * 32, 32)], h0i)
        pltpu.sync_copy(shi.at[pl.ds((even + 1) * 32, 32)], h1i)
        for g in range(2):
            m0 = h0f[pl.ds(g * 16, 16)]
            m1 = h1f[pl.ds(g * 16, 16)]
            i0 = h0i[pl.ds(g * 16, 16)]
            i1 = h1i[pl.ds(g * 16, 16)]
            mrg[pl.ds(g * 16, 16)] = jnp.where(m0 >= m1, i0, i1)

        lov = jnp.full((16,), lo, jnp.int32)
        hiv = jnp.full((16,), lo + HALF, jnp.int32)
        two = _splatf(2.0)

        def force_body(t, _):
            iv = plsc.load_gather(mrg, [jnp.full((16,), t, jnp.int32)])
            mk = jnp.logical_and(lane0,
                 jnp.logical_and(iv >= lov, iv < hiv))
            plsc.store_scatter(bto_v, [iv], two, mask=mk)
            plsc.store_scatter(bti_v, [iv], jnp.full((16,), t, jnp.int32), mask=mk)
            return 0

        lax.fori_loop(0, G, force_body, 0)

        def issue_p2(cl, slot):
            fs = fbufs[slot]
            hs = [pltpu.async_copy(
                priors_f.at[pl.ds(p * P + lo + cl * CH, CH)],
                fs.at[pl.ds(p * CH, CH)], sems[slot]) for p in range(4)]
            hs += [pltpu.async_copy(
                loc_f.at[pl.ds(b * (4 * P) + j * P + lo + cl * CH, CH)],
                fs.at[pl.ds((4 + j) * CH, CH)], sems[slot]) for j in range(4)]
            hs += [pltpu.async_copy(
                conf_f.at[pl.ds(b * (2 * P) + j * P + lo + cl * CH, CH)],
                fs.at[pl.ds((8 + j) * CH, CH)], sems[slot]) for j in range(2)]
            hs += [pltpu.async_copy(
                landm_f.at[pl.ds(b * (8 * P) + j * P + lo + cl * CH, CH)],
                fs.at[pl.ds((10 + j) * CH, CH)], sems[slot]) for j in range(8)]
            return hs

        accs = (zerof, zerof, zerof, zeroi)
        pend = [issue_p2(0, 0), None]
        for cl in range(NCHH):
            slot = cl % 2
            for hh in pend[slot]:
                hh.wait()
            if cl + 1 < NCHH:
                pend[1 - slot] = issue_p2(cl + 1, 1 - slot)
            base = lo + cl * CH
            fb = fbufs[slot]

            def p2_body(v, carry, fb=fb, base=base):
                al, am, ac, an = carry
                off = v * 16
                gsl = pl.ds(base + off, 16)
                pcx = fb[pl.ds(0 * CH + off, 16)]
                pcy = fb[pl.ds(1 * CH + off, 16)]
                pw = fb[pl.ds(2 * CH + off, 16)]
                ph = fb[pl.ds(3 * CH + off, 16)]
                bto = bto_v[gsl]
                bti = bti_v[gsl]
                pos = bto >= THR
                posf = jnp.where(pos, 1.0, 0.0)
                bt13 = bti * 13
                tx1 = plsc.load_gather(traw, [bt13 + _splati(0)])
                ty1 = plsc.load_gather(traw, [bt13 + _splati(1)])
                tx2 = plsc.load_gather(traw, [bt13 + _splati(2)])
                ty2 = plsc.load_gather(traw, [bt13 + _splati(3)])
                dx = pw * 0.1
                dy = ph * 0.1
                g0 = ((tx1 + tx2) * 0.5 - pcx) / dx
                g1 = ((ty1 + ty2) * 0.5 - pcy) / dy
                g2 = _ln((tx2 - tx1) / pw) * 5.0
                g3 = _ln((ty2 - ty1) / ph) * 5.0
                al = al + (_smooth_l1(fb[pl.ds(4 * CH + off, 16)], g0)
                           + _smooth_l1(fb[pl.ds(5 * CH + off, 16)], g1)
                           + _smooth_l1(fb[pl.ds(6 * CH + off, 16)], g2)
                           + _smooth_l1(fb[pl.ds(7 * CH + off, 16)], g3)) * posf
                lacc = zerof
                for j in range(8):
                    lmp = plsc.load_gather(traw, [bt13 + _splati(4 + j)])
                    if j % 2 == 0:
                        gl = (lmp - pcx) / dx
                    else:
                        gl = (lmp - pcy) / dy
                    lacc = lacc + _smooth_l1(fb[pl.ds((10 + j) * CH + off, 16)], gl)
                am = am + lacc * posf
                c0 = fb[pl.ds(8 * CH + off, 16)]
                c1 = fb[pl.ds(9 * CH + off, 16)]
                mx = jnp.maximum(c0, c1)
                mn = jnp.minimum(c0, c1)
                lse = mx + _ln(1.0 + jnp.exp(mn - mx))
                raw = lse - jnp.where(pos, c1, c0)
                ac = ac + raw * posf
                an = an + jnp.where(pos, 1, 0)
                mined_v[gsl] = lax.bitcast_convert_type(
                    jnp.where(pos, 0.0, raw), jnp.int32)
                return al, am, ac, an

            accs = lax.fori_loop(0, VPC, p2_body, accs)
        al, am, ac, an = accs

        np_loc = jnp.sum(an)
        np_all = np_loc + xchg_count(np_loc, 0)
        k = jnp.minimum(np_all * 7, P - 1)

        def count_gt(x):
            xs = jnp.full((16,), x, jnp.int32)

            def cb(i, acc):
                for u in range(15):
                    vv = mined_v[pl.ds(lo + (i * 15 + u) * 16, 16)]
                    acc = acc + jnp.where(vv > xs, 1, 0)
                return acc

            return jnp.sum(lax.fori_loop(0, HALF // 240, cb, zeroi))

        f0 = count_gt(0)
        f0 = f0 + xchg_count(f0, 1)

        def bis(i, ans):
            cand = ans | (1 << (30 - i))
            cl = count_gt(cand)
            cnt = cl + xchg_count(cl, 2 + i)
            return jnp.where(cnt >= k, cand, ans)

        ans = lax.fori_loop(0, 31, bis, 0)
        tbits = jnp.where(f0 >= k, ans + 1, 0)
        cfin = count_gt(tbits)
        call = cfin + xchg_count(cfin, 33)
        ts_ = jnp.full((16,), tbits, jnp.int32)

        def fin(i, sacc):
            for u in range(15):
                vv = mined_v[pl.ds(lo + (i * 15 + u) * 16, 16)]
                mm = vv > ts_
                val = lax.bitcast_convert_type(vv, jnp.float32)
                sacc = sacc + jnp.where(mm, val, 0.0)
            return sacc

        sacc = lax.fori_loop(0, HALF // 240, fin, zerof)
        tval = jnp.max(lax.bitcast_convert_type(ts_, jnp.float32))
        rem = jnp.where(h == 0, (k - call).astype(jnp.float32) * tval, 0.0)
        loss_c = jnp.sum(ac) + jnp.sum(sacc) + rem

        outv = jnp.where(ivec == 0, jnp.sum(al),
               jnp.where(ivec == 1, loss_c,
               jnp.where(ivec == 2, jnp.sum(am), np_loc.astype(jnp.float32))))
        outbuf[...] = outv
        pltpu.sync_copy(outbuf, out.at[w])

    return sc_loss


_SC_LOSS = _make_sc_kernel()


def kernel(loc_data, conf_data, landm_data, priors, targets):
    Bn, Pn, _ = loc_data.shape
    priors_f = priors.T.reshape(-1)
    loc_f = loc_data.transpose(0, 2, 1).reshape(-1)
    conf_f = conf_data.transpose(0, 2, 1).reshape(-1)
    landm_f = landm_data.transpose(0, 2, 1).reshape(-1)
    targ_f = targets.reshape(-1)
    part = _SC_LOSS(priors_f, loc_f, conf_f, landm_f, targ_f)
    n = jnp.maximum(jnp.sum(part[:, 3]), 1.0)
    return part[:, 0].sum() / n, part[:, 1].sum() / n, part[:, 2].sum() / n
